# baseline (speedup 1.0000x reference)
import jax
import jax.numpy as jnp
from jax import lax
from jax.experimental import pallas as pl
from jax.experimental.pallas import tpu as pltpu
from jax.experimental.pallas import tpu_sc as plsc

_NC, _NS = 2, 16
_CB = 8


def _compute_chunk(zb, qb, ib, h_count):
    def row(h, carry):
        for j in range(2):
            sl = pl.ds(16 * j, 16)
            acc = None
            for d in range(8):
                x = zb[d, h, sl]
                y = 7.0 / (1.0 + jnp.exp(x * -2.0))
                idx = (y + 0.5).astype(jnp.int32)
                qb[d, h, sl] = idx.astype(jnp.float32) * (2.0 / 7.0) - 1.0
                acc = idx if acc is None else (acc << 3) | idx
            ib[h, sl] = acc
        return carry

    lax.fori_loop(0, h_count, row, 0)


def _sc_body(z_hbm, q_hbm, i_hbm, zb, qb, ib, zsems, qsems, isems):
    b = lax.axis_index("c") * _NS + lax.axis_index("s")

    def start_in(c, par):
        pltpu.make_async_copy(
            z_hbm.at[b, pl.ds(c * 8, 8)], zb.at[par], zsems[par]
        ).start()

    def wait_in(par):
        pltpu.make_async_copy(
            z_hbm.at[b, pl.ds(0, 8)], zb.at[par], zsems[par]
        ).wait()

    def start_out(c, par):
        pltpu.make_async_copy(
            qb.at[par], q_hbm.at[b, pl.ds(c * 8, 8)], qsems[par]
        ).start()
        pltpu.make_async_copy(ib.at[par], i_hbm.at[b, c], isems[par]).start()

    def wait_out(par):
        pltpu.make_async_copy(
            qb.at[par], q_hbm.at[b, pl.ds(0, 8)], qsems[par]
        ).wait()
        pltpu.make_async_copy(ib.at[par], i_hbm.at[b, 0], isems[par]).wait()

    h = zb.shape[2]
    start_in(0, 0)
    for c in range(_CB):
        par = c % 2
        if c + 1 < _CB:
            start_in(c + 1, (c + 1) % 2)
        wait_in(par)
        if c >= 2:
            wait_out(par)
        _compute_chunk(zb.at[par], qb.at[par], ib.at[par], h)
        start_out(c, par)
    wait_out(0)
    wait_out(1)


def kernel(z):
    B, D, H, W = z.shape
    mesh = plsc.VectorSubcoreMesh(
        core_axis_name="c", subcore_axis_name="s",
        num_cores=_NC, num_subcores=_NS,
    )
    kfn = pl.kernel(
        _sc_body,
        out_type=(
            jax.ShapeDtypeStruct((B, D, H, W), jnp.float32),
            jax.ShapeDtypeStruct((B, D // _CB, H, W), jnp.int32),
        ),
        mesh=mesh,
        scratch_types=[
            pltpu.VMEM((2, 8, H, W), jnp.float32),
            pltpu.VMEM((2, 8, H, W), jnp.float32),
            pltpu.VMEM((2, H, W), jnp.int32),
            (pltpu.SemaphoreType.DMA, pltpu.SemaphoreType.DMA),
            (pltpu.SemaphoreType.DMA, pltpu.SemaphoreType.DMA),
            (pltpu.SemaphoreType.DMA, pltpu.SemaphoreType.DMA),
        ],
        compiler_params=pltpu.CompilerParams(use_tc_tiling_on_sc=False),
    )
    return kfn(z)
